# baseline (speedup 1.0000x reference)
import functools

import jax
import jax.numpy as jnp
from jax import lax
from jax.experimental import pallas as pl
from jax.experimental.pallas import tpu as pltpu
from jax.experimental.pallas import tpu_sc as plsc

K = 32
PAD_COLS = 528
OUT_COLS = 516

NEG_INF = float("-inf")


def _table_body(pos_ref, feat_ref, inv_ref, tab_ref):
    pos = pos_ref[...]
    feat = feat_ref[...]
    cx = 0.5 * (pos[:, 0:1] + pos[:, 2:3])
    cy = 0.5 * (pos[:, 1:2] + pos[:, 3:4])
    w = jnp.maximum(pos[:, 2:3] - pos[:, 0:1], 1e-6)
    h = jnp.maximum(pos[:, 3:4] - pos[:, 1:2], 1e-6)
    tab_ref[:, 0:1] = cx * inv_ref[0, 0]
    tab_ref[:, 1:2] = cy * inv_ref[0, 1]
    tab_ref[:, 2:3] = jnp.log(w)
    tab_ref[:, 3:4] = jnp.log(h)
    tab_ref[:, 4:4 + feat.shape[1]] = feat
    tab_ref[:, 4 + feat.shape[1]:] = jnp.zeros(
        (feat.shape[0], PAD_COLS - 4 - feat.shape[1]), jnp.float32)


def _build_table(pos, feat, inv_wh, block_rows):
    n, dim = feat.shape
    grid = (n // block_rows,)
    return pl.pallas_call(
        _table_body,
        grid=grid,
        in_specs=[
            pl.BlockSpec((block_rows, 4), lambda i: (i, 0)),
            pl.BlockSpec((block_rows, dim), lambda i: (i, 0)),
            pl.BlockSpec(memory_space=pltpu.SMEM),
        ],
        out_specs=pl.BlockSpec((block_rows, PAD_COLS), lambda i: (i, 0)),
        out_shape=jax.ShapeDtypeStruct((n, PAD_COLS), jnp.float32),
    )(pos, feat, inv_wh)


def _normalize_body(feat_ref, out_ref):
    f = feat_ref[...]
    nrm = jnp.sqrt(jnp.sum(f * f, axis=1, keepdims=True))
    out_ref[...] = f / jnp.maximum(nrm, 1e-12)


def _normalize_rows(feat, block_rows):
    n, dim = feat.shape
    return pl.pallas_call(
        _normalize_body,
        grid=(n // block_rows,),
        in_specs=[pl.BlockSpec((block_rows, dim), lambda i: (i, 0))],
        out_specs=pl.BlockSpec((block_rows, dim), lambda i: (i, 0)),
        out_shape=jax.ShapeDtypeStruct((n, dim), jnp.float32),
    )(feat)


def _topk_body(tf_ref, dq_ref, out_ref, vals_ref, idxs_ref, *, bd, nj, k):
    j = pl.program_id(1)

    @pl.when(j == 0)
    def _():
        vals_ref[...] = jnp.full(vals_ref.shape, NEG_INF, jnp.float32)
        idxs_ref[...] = jnp.zeros(idxs_ref.shape, jnp.int32)

    s = lax.dot_general(
        tf_ref[...], dq_ref[...],
        dimension_numbers=(((1,), (1,)), ((), ())),
        preferred_element_type=jnp.float32,
        precision=lax.Precision.DEFAULT,
    )
    bt = s.shape[0]
    colk = lax.broadcasted_iota(jnp.int32, (bt, k), 1)
    big = jnp.int32(2147483647)

    def block_col():
        return lax.broadcasted_iota(jnp.int32, (bt, bd), 1) + j * bd

    tau = vals_ref[:, k - 1:k]
    cnt = jnp.sum((s > tau).astype(jnp.int32), axis=1, keepdims=True)
    r = jnp.minimum(jnp.max(cnt), k)

    def extract_block(i, carry):
        sblk, bvals, bidx = carry
        m = jnp.max(sblk, axis=1, keepdims=True)
        picked = jnp.min(jnp.where(sblk == m, block_col(), big),
                         axis=1, keepdims=True)
        sblk = jnp.where(block_col() == picked, NEG_INF, sblk)
        bvals = jnp.where(colk == i, m, bvals)
        bidx = jnp.where(colk == i, picked, bidx)
        return sblk, bvals, bidx

    _, bvals, bidx = lax.fori_loop(
        0, r, extract_block,
        (s, jnp.full((bt, k), NEG_INF, jnp.float32),
         jnp.zeros((bt, k), jnp.int32)))

    scat0 = jnp.concatenate([vals_ref[...], bvals], axis=1)
    icat = jnp.concatenate([idxs_ref[...], bidx], axis=1)

    def extract_merge(i, carry):
        scat, vacc, iacc = carry
        m = jnp.max(scat, axis=1, keepdims=True)
        is_max = scat == m
        picked = jnp.min(jnp.where(is_max, icat, big), axis=1, keepdims=True)
        scat = jnp.where(is_max & (icat == picked), NEG_INF, scat)
        vacc = jnp.where(colk == i, m, vacc)
        iacc = jnp.where(colk == i, picked, iacc)
        return scat, vacc, iacc

    _, vals, idxs = lax.fori_loop(
        0, k, extract_merge,
        (scat0, jnp.zeros((bt, k), jnp.float32), jnp.zeros((bt, k), jnp.int32)))
    vals_ref[...] = vals
    idxs_ref[...] = idxs

    @pl.when(j == nj - 1)
    def _():
        out_ref[...] = idxs_ref[...]


def _topk_indices(tfeat, dqn, bt, bd, k):
    t, dim = tfeat.shape
    d = dqn.shape[0]
    nj = d // bd
    return pl.pallas_call(
        functools.partial(_topk_body, bd=bd, nj=nj, k=k),
        grid=(t // bt, nj),
        in_specs=[
            pl.BlockSpec((bt, dim), lambda i, j: (i, 0)),
            pl.BlockSpec((bd, dim), lambda i, j: (j, 0)),
        ],
        out_specs=pl.BlockSpec((bt, k), lambda i, j: (i, 0)),
        out_shape=jax.ShapeDtypeStruct((t, k), jnp.int32),
        scratch_shapes=[
            pltpu.VMEM((bt, k), jnp.float32),
            pltpu.VMEM((bt, k), jnp.int32),
        ],
    )(tfeat, dqn)


def _edge_kernel(t_total, k, n_half):
    info = plsc.get_sparse_core_info()
    nw = info.num_cores * info.num_subcores
    t_per_w = t_total // nw
    mesh = plsc.VectorSubcoreMesh(core_axis_name="c", subcore_axis_name="s")
    chunk_offs = [c * 16 for c in range(OUT_COLS // 16)] + [OUT_COLS - 16]

    @functools.partial(
        pl.kernel, mesh=mesh,
        compiler_params=pltpu.CompilerParams(
            use_tc_tiling_on_sc=False, needs_layout_passes=False),
        out_type=jax.ShapeDtypeStruct((2 * n_half, OUT_COLS), jnp.float32),
        scratch_types=[
            pltpu.VMEM((t_per_w * k,), jnp.int32),
            pltpu.VMEM((2, PAD_COLS), jnp.float32),
            pltpu.VMEM((k, PAD_COLS), jnp.float32),
            pltpu.VMEM((k, PAD_COLS), jnp.float32),
            pltpu.VMEM((k, OUT_COLS), jnp.float32),
            pltpu.VMEM((k, OUT_COLS), jnp.float32),
            pltpu.VMEM((k, OUT_COLS), jnp.float32),
            pltpu.VMEM((k, OUT_COLS), jnp.float32),
            pltpu.SemaphoreType.DMA,
            pltpu.SemaphoreType.DMA,
            pltpu.SemaphoreType.DMA,
        ],
    )
    def edge(idx_hbm, atab_hbm, btab_hbm, out_hbm,
             idx_all, a01, b0, b1, o1a, o2a, o1b, o2b, gsem0, gsem1, osem):
        wid = lax.axis_index("s") * info.num_cores + lax.axis_index("c")
        base = wid * t_per_w
        lane = lax.broadcasted_iota(jnp.int32, (16,), 0)
        is_pos = lane < 4
        pltpu.sync_copy(idx_hbm.at[pl.ds(base * k, t_per_w * k)], idx_all)

        def compute(par, b_rows, out1, out2):
            def per_edge(j, _):
                jv = jnp.full((16,), j, jnp.int32)
                for off in chunk_offs:
                    cols = lane + off
                    av = a01[par, pl.ds(off, 16)]
                    bv = plsc.load_gather(b_rows, [jv, cols])
                    f = 0.5 * (av + bv)
                    if off == 0:
                        diff = av - bv
                        o1 = jnp.where(is_pos, diff, f)
                        o2 = jnp.where(is_pos, -diff, f)
                    else:
                        o1 = f
                        o2 = f
                    plsc.store_scatter(out1, [jv, cols], o1)
                    plsc.store_scatter(out2, [jv, cols], o2)
                return 0

            lax.fori_loop(0, k, per_edge, 0)

        def per_pair(g, _):
            t0 = base + 2 * g
            cp_g0 = pltpu.async_copy(
                btab_hbm.at[idx_all.at[pl.ds(2 * g * k, k)]], b0, gsem0)
            cp_g1 = pltpu.async_copy(
                btab_hbm.at[idx_all.at[pl.ds((2 * g + 1) * k, k)]], b1, gsem1)
            pltpu.sync_copy(atab_hbm.at[pl.ds(t0, 2)], a01)
            cp_g0.wait()
            compute(0, b0, o1a, o2a)
            cp_o1 = pltpu.async_copy(o1a, out_hbm.at[pl.ds(t0 * k, k)], osem)
            cp_o2 = pltpu.async_copy(
                o2a, out_hbm.at[pl.ds(n_half + t0 * k, k)], osem)
            cp_g1.wait()
            compute(1, b1, o1b, o2b)
            cp_o3 = pltpu.async_copy(
                o1b, out_hbm.at[pl.ds((t0 + 1) * k, k)], osem)
            cp_o4 = pltpu.async_copy(
                o2b, out_hbm.at[pl.ds(n_half + (t0 + 1) * k, k)], osem)
            cp_o1.wait()
            cp_o2.wait()
            cp_o3.wait()
            cp_o4.wait()
            return 0

        lax.fori_loop(0, t_per_w // 2, per_pair, 0)

    return edge


def kernel(tracklet_feat, det_feat, tracklet_pos, det_pos, img_w, img_h):
    t, dim = tracklet_feat.shape
    d = det_feat.shape[0]
    k = min(K, d)
    inv_wh = jnp.stack([
        1.0 / jnp.asarray(img_w, jnp.float32),
        1.0 / jnp.asarray(img_h, jnp.float32),
    ]).reshape(1, 2)

    atab = _build_table(tracklet_pos, tracklet_feat, inv_wh, 512)
    btab = _build_table(det_pos, det_feat, inv_wh, 1024)
    tqn = _normalize_rows(tracklet_feat, 512)
    dqn = _normalize_rows(det_feat, 1024)
    idx = _topk_indices(tqn, dqn, 512, 4096, k)

    n_half = t * k
    return _edge_kernel(t, k, n_half)(idx.reshape(-1), atab, btab)


# speedup vs baseline: 1.0465x; 1.0465x over previous
import functools

import jax
import jax.numpy as jnp
from jax import lax
from jax.experimental import pallas as pl
from jax.experimental.pallas import tpu as pltpu
from jax.experimental.pallas import tpu_sc as plsc

K = 32
PAD_COLS = 528
OUT_COLS = 516

NEG_INF = float("-inf")


def _table_body(pos_ref, feat_ref, inv_ref, tab_ref):
    pos = pos_ref[...]
    feat = feat_ref[...]
    cx = 0.5 * (pos[:, 0:1] + pos[:, 2:3])
    cy = 0.5 * (pos[:, 1:2] + pos[:, 3:4])
    w = jnp.maximum(pos[:, 2:3] - pos[:, 0:1], 1e-6)
    h = jnp.maximum(pos[:, 3:4] - pos[:, 1:2], 1e-6)
    tab_ref[:, 0:1] = cx * inv_ref[0, 0]
    tab_ref[:, 1:2] = cy * inv_ref[0, 1]
    tab_ref[:, 2:3] = jnp.log(w)
    tab_ref[:, 3:4] = jnp.log(h)
    tab_ref[:, 4:4 + feat.shape[1]] = feat
    tab_ref[:, 4 + feat.shape[1]:] = jnp.zeros(
        (feat.shape[0], PAD_COLS - 4 - feat.shape[1]), jnp.float32)


def _build_table(pos, feat, inv_wh, block_rows):
    n, dim = feat.shape
    grid = (n // block_rows,)
    return pl.pallas_call(
        _table_body,
        grid=grid,
        in_specs=[
            pl.BlockSpec((block_rows, 4), lambda i: (i, 0)),
            pl.BlockSpec((block_rows, dim), lambda i: (i, 0)),
            pl.BlockSpec(memory_space=pltpu.SMEM),
        ],
        out_specs=pl.BlockSpec((block_rows, PAD_COLS), lambda i: (i, 0)),
        out_shape=jax.ShapeDtypeStruct((n, PAD_COLS), jnp.float32),
    )(pos, feat, inv_wh)


def _normalize_body(feat_ref, out_ref):
    f = feat_ref[...]
    nrm = jnp.sqrt(jnp.sum(f * f, axis=1, keepdims=True))
    out_ref[...] = f / jnp.maximum(nrm, 1e-12)


def _normalize_rows(feat, block_rows):
    n, dim = feat.shape
    return pl.pallas_call(
        _normalize_body,
        grid=(n // block_rows,),
        in_specs=[pl.BlockSpec((block_rows, dim), lambda i: (i, 0))],
        out_specs=pl.BlockSpec((block_rows, dim), lambda i: (i, 0)),
        out_shape=jax.ShapeDtypeStruct((n, dim), jnp.float32),
    )(feat)


def _topk_body(tf_ref, dq_ref, out_ref, vals_ref, idxs_ref, *, bd, nj, k):
    j = pl.program_id(1)

    @pl.when(j == 0)
    def _():
        vals_ref[...] = jnp.full(vals_ref.shape, NEG_INF, jnp.float32)
        idxs_ref[...] = jnp.zeros(idxs_ref.shape, jnp.int32)

    s = lax.dot_general(
        tf_ref[...], dq_ref[...],
        dimension_numbers=(((1,), (1,)), ((), ())),
        preferred_element_type=jnp.float32,
        precision=lax.Precision.DEFAULT,
    )
    bt = s.shape[0]
    colk = lax.broadcasted_iota(jnp.int32, (bt, k), 1)
    big = jnp.int32(2147483647)

    def block_col():
        return lax.broadcasted_iota(jnp.int32, (bt, bd), 1) + j * bd

    tau = vals_ref[:, k - 1:k]
    cnt = jnp.sum((s > tau).astype(jnp.int32), axis=1, keepdims=True)
    r = jnp.minimum(jnp.max(cnt), k)

    def extract_block(i, carry):
        sblk, bvals, bidx = carry
        m = jnp.max(sblk, axis=1, keepdims=True)
        picked = jnp.min(jnp.where(sblk == m, block_col(), big),
                         axis=1, keepdims=True)
        sblk = jnp.where(block_col() == picked, NEG_INF, sblk)
        bvals = jnp.where(colk == i, m, bvals)
        bidx = jnp.where(colk == i, picked, bidx)
        return sblk, bvals, bidx

    _, bvals, bidx = lax.fori_loop(
        0, r, extract_block,
        (s, jnp.full((bt, k), NEG_INF, jnp.float32),
         jnp.zeros((bt, k), jnp.int32)))

    scat0 = jnp.concatenate([vals_ref[...], bvals], axis=1)
    icat = jnp.concatenate([idxs_ref[...], bidx], axis=1)

    def extract_merge(i, carry):
        scat, vacc, iacc = carry
        m = jnp.max(scat, axis=1, keepdims=True)
        is_max = scat == m
        picked = jnp.min(jnp.where(is_max, icat, big), axis=1, keepdims=True)
        scat = jnp.where(is_max & (icat == picked), NEG_INF, scat)
        vacc = jnp.where(colk == i, m, vacc)
        iacc = jnp.where(colk == i, picked, iacc)
        return scat, vacc, iacc

    _, vals, idxs = lax.fori_loop(
        0, k, extract_merge,
        (scat0, jnp.zeros((bt, k), jnp.float32), jnp.zeros((bt, k), jnp.int32)))
    vals_ref[...] = vals
    idxs_ref[...] = idxs

    @pl.when(j == nj - 1)
    def _():
        out_ref[...] = idxs_ref[...]


def _topk_indices(tfeat, dqn, bt, bd, k):
    t, dim = tfeat.shape
    d = dqn.shape[0]
    nj = d // bd
    return pl.pallas_call(
        functools.partial(_topk_body, bd=bd, nj=nj, k=k),
        grid=(t // bt, nj),
        in_specs=[
            pl.BlockSpec((bt, dim), lambda i, j: (i, 0)),
            pl.BlockSpec((bd, dim), lambda i, j: (j, 0)),
        ],
        out_specs=pl.BlockSpec((bt, k), lambda i, j: (i, 0)),
        out_shape=jax.ShapeDtypeStruct((t, k), jnp.int32),
        scratch_shapes=[
            pltpu.VMEM((bt, k), jnp.float32),
            pltpu.VMEM((bt, k), jnp.int32),
        ],
    )(tfeat, dqn)


def _edge_kernel(t_total, k, n_half):
    info = plsc.get_sparse_core_info()
    nw = info.num_cores * info.num_subcores
    t_per_w = t_total // nw
    mesh = plsc.VectorSubcoreMesh(core_axis_name="c", subcore_axis_name="s")
    chunk_offs = [c * 16 for c in range(OUT_COLS // 16)] + [OUT_COLS - 16]

    @functools.partial(
        pl.kernel, mesh=mesh,
        compiler_params=pltpu.CompilerParams(
            use_tc_tiling_on_sc=False, needs_layout_passes=False),
        out_type=jax.ShapeDtypeStruct((2 * n_half, OUT_COLS), jnp.float32),
        scratch_types=[
            pltpu.VMEM((t_per_w * k,), jnp.int32),
            pltpu.VMEM((2, PAD_COLS), jnp.float32),
            pltpu.VMEM((k, PAD_COLS), jnp.float32),
            pltpu.VMEM((k, PAD_COLS), jnp.float32),
            pltpu.VMEM((k, OUT_COLS), jnp.float32),
            pltpu.VMEM((k, OUT_COLS), jnp.float32),
            pltpu.VMEM((k, OUT_COLS), jnp.float32),
            pltpu.VMEM((k, OUT_COLS), jnp.float32),
            pltpu.SemaphoreType.DMA,
            pltpu.SemaphoreType.DMA,
            pltpu.SemaphoreType.DMA,
        ],
    )
    def edge(idx_hbm, atab_hbm, btab_hbm, out_hbm,
             idx_all, a01, b0, b1, o1a, o2a, o1b, o2b, gsem0, gsem1, osem):
        wid = lax.axis_index("s") * info.num_cores + lax.axis_index("c")
        base = wid * t_per_w
        lane = lax.broadcasted_iota(jnp.int32, (16,), 0)
        is_pos = lane < 4
        pltpu.sync_copy(idx_hbm.at[pl.ds(base * k, t_per_w * k)], idx_all)

        def compute(par, b_rows, out1, out2):
            def per_edge(j, _):
                jv = jnp.full((16,), j, jnp.int32)
                for off in chunk_offs:
                    cols = lane + off
                    av = a01[par, pl.ds(off, 16)]
                    bv = plsc.load_gather(b_rows, [jv, cols])
                    f = 0.5 * (av + bv)
                    if off == 0:
                        diff = av - bv
                        o1 = jnp.where(is_pos, diff, f)
                        o2 = jnp.where(is_pos, -diff, f)
                    else:
                        o1 = f
                        o2 = f
                    plsc.store_scatter(out1, [jv, cols], o1)
                    plsc.store_scatter(out2, [jv, cols], o2)
                return 0

            lax.fori_loop(0, k, per_edge, 0)

        def drain_outs(t0):
            pltpu.make_async_copy(o1a, out_hbm.at[pl.ds(t0 * k, k)], osem).wait()
            pltpu.make_async_copy(
                o2a, out_hbm.at[pl.ds(n_half + t0 * k, k)], osem).wait()
            pltpu.make_async_copy(
                o1b, out_hbm.at[pl.ds((t0 + 1) * k, k)], osem).wait()
            pltpu.make_async_copy(
                o2b, out_hbm.at[pl.ds(n_half + (t0 + 1) * k, k)], osem).wait()

        def per_pair(g, _):
            t0 = base + 2 * g
            cp_g0 = pltpu.async_copy(
                btab_hbm.at[idx_all.at[pl.ds(2 * g * k, k)]], b0, gsem0)
            cp_g1 = pltpu.async_copy(
                btab_hbm.at[idx_all.at[pl.ds((2 * g + 1) * k, k)]], b1, gsem1)
            pltpu.sync_copy(atab_hbm.at[pl.ds(t0, 2)], a01)

            @pl.when(g > 0)
            def _():
                drain_outs(t0)

            cp_g0.wait()
            compute(0, b0, o1a, o2a)
            pltpu.async_copy(o1a, out_hbm.at[pl.ds(t0 * k, k)], osem)
            pltpu.async_copy(o2a, out_hbm.at[pl.ds(n_half + t0 * k, k)], osem)
            cp_g1.wait()
            compute(1, b1, o1b, o2b)
            pltpu.async_copy(o1b, out_hbm.at[pl.ds((t0 + 1) * k, k)], osem)
            pltpu.async_copy(
                o2b, out_hbm.at[pl.ds(n_half + (t0 + 1) * k, k)], osem)
            return 0

        lax.fori_loop(0, t_per_w // 2, per_pair, 0)
        drain_outs(base)

    return edge


def kernel(tracklet_feat, det_feat, tracklet_pos, det_pos, img_w, img_h):
    t, dim = tracklet_feat.shape
    d = det_feat.shape[0]
    k = min(K, d)
    inv_wh = jnp.stack([
        1.0 / jnp.asarray(img_w, jnp.float32),
        1.0 / jnp.asarray(img_h, jnp.float32),
    ]).reshape(1, 2)

    atab = _build_table(tracklet_pos, tracklet_feat, inv_wh, 512)
    btab = _build_table(det_pos, det_feat, inv_wh, 1024)
    tqn = _normalize_rows(tracklet_feat, 512)
    dqn = _normalize_rows(det_feat, 1024)
    idx = _topk_indices(tqn, dqn, 512, 2048, k)

    n_half = t * k
    return _edge_kernel(t, k, n_half)(idx.reshape(-1), atab, btab)
